# baseline (speedup 1.0000x reference)
import functools

import jax
import jax.numpy as jnp
from jax import lax
from jax.experimental import pallas as pl
from jax.experimental.pallas import tpu as pltpu
from jax.experimental.pallas import tpu_sc as plsc

_MAX_LEN = 1000
_D = 128
_B = 16384
_NC = 2
_NS = 16
_NW = _NC * _NS
_BPW = _B // _NW
_CHUNK = 64
_NCHUNK = _BPW // _CHUNK
_TROWS = 64
_TREM = _MAX_LEN - _TROWS * (_NS - 1)


_NHBM = 2


def _scratch_types():
    return [
        pltpu.VMEM_SHARED((_MAX_LEN, _D), jnp.float32),
        pltpu.VMEM((_NCHUNK, _CHUNK), jnp.int32),
        pltpu.VMEM((_BPW, _D), jnp.float32),
        *([pltpu.SemaphoreType.DMA] * _NCHUNK),
        pltpu.SemaphoreType.DMA,
        pltpu.SemaphoreType.DMA,
    ]


def _make_gather():
    mesh = plsc.VectorSubcoreMesh(core_axis_name="c", subcore_axis_name="s")

    @functools.partial(
        pl.kernel,
        mesh=mesh,
        out_type=jax.ShapeDtypeStruct((_NW, _BPW, _D), jnp.float32),
        scratch_types=_scratch_types(),
    )
    def gather_kernel(table_hbm, idx_hbm, out_hbm, table_sp, idx_v, rows_v, *sems):
        gsems, ssem, tsem = sems[:_NCHUNK], sems[_NCHUNK], sems[_NCHUNK + 1]
        cid = lax.axis_index("c")
        sid = lax.axis_index("s")
        wid = sid * _NC + cid
        @pl.when(sid < _NS - 1)
        def _():
            pltpu.make_async_copy(
                table_hbm.at[pl.ds(sid * _TROWS, _TROWS)],
                table_sp.at[pl.ds(sid * _TROWS, _TROWS)],
                tsem,
            ).start()

        @pl.when(sid == _NS - 1)
        def _():
            pltpu.make_async_copy(
                table_hbm.at[pl.ds((_NS - 1) * _TROWS, _TREM)],
                table_sp.at[pl.ds((_NS - 1) * _TROWS, _TREM)],
                tsem,
            ).start()

        idx_cp = pltpu.make_async_copy(idx_hbm.at[wid], idx_v, ssem)
        idx_cp.start()
        idx_cp.wait()
        gathers = []
        for j in range(_NHBM):
            g = pltpu.make_async_copy(
                table_hbm.at[idx_v.at[j]],
                rows_v.at[pl.ds(j * _CHUNK, _CHUNK)],
                gsems[j],
            )
            g.start()
            gathers.append(g)
        @pl.when(sid < _NS - 1)
        def _():
            pltpu.make_async_copy(
                table_hbm.at[pl.ds(sid * _TROWS, _TROWS)],
                table_sp.at[pl.ds(sid * _TROWS, _TROWS)],
                tsem,
            ).wait()

        @pl.when(sid == _NS - 1)
        def _():
            pltpu.make_async_copy(
                table_hbm.at[pl.ds((_NS - 1) * _TROWS, _TREM)],
                table_sp.at[pl.ds((_NS - 1) * _TROWS, _TREM)],
                tsem,
            ).wait()

        plsc.subcore_barrier()
        for j in range(_NHBM, _NCHUNK):
            g = pltpu.make_async_copy(
                table_sp.at[idx_v.at[j]],
                rows_v.at[pl.ds(j * _CHUNK, _CHUNK)],
                gsems[j],
            )
            g.start()
            gathers.append(g)
        stores = []
        for j in range(_NCHUNK):
            gathers[j].wait()
            s = pltpu.make_async_copy(
                rows_v.at[pl.ds(j * _CHUNK, _CHUNK)],
                out_hbm.at[wid, pl.ds(j * _CHUNK, _CHUNK)],
                ssem,
            )
            s.start()
            stores.append(s)
        for s in stores:
            s.wait()

    return gather_kernel


_gather = _make_gather()


def kernel(pos_encoding, t):
    idx = jnp.reshape(t.astype(jnp.int32), (_NW, _NCHUNK, _CHUNK))
    out = _gather(pos_encoding, idx)
    return jnp.reshape(out, (_B, _D))


# speedup vs baseline: 1.0223x; 1.0223x over previous
import functools

import jax
import jax.numpy as jnp
from jax import lax
from jax.experimental import pallas as pl
from jax.experimental.pallas import tpu as pltpu
from jax.experimental.pallas import tpu_sc as plsc

_MAX_LEN = 1000
_D = 128
_B = 16384
_NC = 2
_NS = 16
_NW = _NC * _NS
_BPW = _B // _NW
_CHUNK = 64
_NCHUNK = _BPW // _CHUNK
_TROWS = 64
_TREM = _MAX_LEN - _TROWS * (_NS - 1)


def _make_gather():
    mesh = plsc.VectorSubcoreMesh(core_axis_name="c", subcore_axis_name="s")

    @functools.partial(
        pl.kernel,
        mesh=mesh,
        out_type=jax.ShapeDtypeStruct((_NW, _BPW, _D), jnp.float32),
        scratch_types=[
            pltpu.VMEM_SHARED((_MAX_LEN, _D), jnp.float32),
            pltpu.VMEM((_NCHUNK, _CHUNK), jnp.int32),
            pltpu.VMEM((_BPW, _D), jnp.float32),
            *([pltpu.SemaphoreType.DMA] * _NCHUNK),
            pltpu.SemaphoreType.DMA,
        ],
    )
    def gather_kernel(table_hbm, idx_hbm, out_hbm, table_sp, idx_v, rows_v, *sems):
        gsems, ssem = sems[:_NCHUNK], sems[_NCHUNK]
        cid = lax.axis_index("c")
        sid = lax.axis_index("s")
        wid = cid * _NS + sid
        idx_cp = pltpu.make_async_copy(idx_hbm.at[wid], idx_v, ssem)
        idx_cp.start()
        @pl.when(sid < _NS - 1)
        def _():
            pltpu.sync_copy(
                table_hbm.at[pl.ds(sid * _TROWS, _TROWS)],
                table_sp.at[pl.ds(sid * _TROWS, _TROWS)],
            )

        @pl.when(sid == _NS - 1)
        def _():
            pltpu.sync_copy(
                table_hbm.at[pl.ds((_NS - 1) * _TROWS, _TREM)],
                table_sp.at[pl.ds((_NS - 1) * _TROWS, _TREM)],
            )

        idx_cp.wait()
        plsc.subcore_barrier()
        gathers = []
        for j in range(_NCHUNK):
            gathers.append(
                pltpu.make_async_copy(
                    table_sp.at[idx_v.at[j]],
                    rows_v.at[pl.ds(j * _CHUNK, _CHUNK)],
                    gsems[j],
                )
            )
        for c in gathers:
            c.start()
        stores = []
        for j in range(_NCHUNK):
            gathers[j].wait()
            s = pltpu.make_async_copy(
                rows_v.at[pl.ds(j * _CHUNK, _CHUNK)],
                out_hbm.at[wid, pl.ds(j * _CHUNK, _CHUNK)],
                ssem,
            )
            s.start()
            stores.append(s)
        for s in stores:
            s.wait()

    return gather_kernel


_gather = _make_gather()


def kernel(pos_encoding, t):
    idx = jnp.reshape(t.astype(jnp.int32), (_NW, _NCHUNK, _CHUNK))
    out = _gather(pos_encoding, idx)
    return jnp.reshape(out, (_B, _D))


# speedup vs baseline: 1.0236x; 1.0013x over previous
import functools

import jax
import jax.numpy as jnp
from jax import lax
from jax.experimental import pallas as pl
from jax.experimental.pallas import tpu as pltpu
from jax.experimental.pallas import tpu_sc as plsc

_MAX_LEN = 1000
_D = 128
_B = 16384
_NC = 2
_NS = 16
_NW = _NC * _NS
_BPW = _B // _NW
_CHUNK = 64
_NCHUNK = _BPW // _CHUNK
_TROWS = 64
_TREM = _MAX_LEN - _TROWS * (_NS - 1)


def _make_gather():
    mesh = plsc.VectorSubcoreMesh(core_axis_name="c", subcore_axis_name="s")

    @functools.partial(
        pl.kernel,
        mesh=mesh,
        out_type=jax.ShapeDtypeStruct((_NW, _BPW, _D), jnp.float32),
        scratch_types=[
            pltpu.VMEM_SHARED((_MAX_LEN, _D), jnp.float32),
            pltpu.VMEM((_NCHUNK, _CHUNK), jnp.int32),
            pltpu.VMEM((_BPW, _D), jnp.float32),
            *([pltpu.SemaphoreType.DMA] * _NCHUNK),
            pltpu.SemaphoreType.DMA,
        ],
    )
    def gather_kernel(table_hbm, idx_hbm, out_hbm, table_sp, idx_v, rows_v, *sems):
        gsems, ssem = sems[:_NCHUNK], sems[_NCHUNK]
        cid = lax.axis_index("c")
        sid = lax.axis_index("s")
        wid = sid * _NC + cid
        idx_cp = pltpu.make_async_copy(idx_hbm.at[wid], idx_v, ssem)
        idx_cp.start()
        @pl.when(sid < _NS - 1)
        def _():
            pltpu.sync_copy(
                table_hbm.at[pl.ds(sid * _TROWS, _TROWS)],
                table_sp.at[pl.ds(sid * _TROWS, _TROWS)],
            )

        @pl.when(sid == _NS - 1)
        def _():
            pltpu.sync_copy(
                table_hbm.at[pl.ds((_NS - 1) * _TROWS, _TREM)],
                table_sp.at[pl.ds((_NS - 1) * _TROWS, _TREM)],
            )

        idx_cp.wait()
        plsc.subcore_barrier()
        gathers = []
        for j in range(_NCHUNK):
            gathers.append(
                pltpu.make_async_copy(
                    table_sp.at[idx_v.at[j]],
                    rows_v.at[pl.ds(j * _CHUNK, _CHUNK)],
                    gsems[j],
                )
            )
        for c in gathers:
            c.start()
        stores = []
        for j in range(_NCHUNK):
            gathers[j].wait()
            s = pltpu.make_async_copy(
                rows_v.at[pl.ds(j * _CHUNK, _CHUNK)],
                out_hbm.at[wid, pl.ds(j * _CHUNK, _CHUNK)],
                ssem,
            )
            s.start()
            stores.append(s)
        for s in stores:
            s.wait()

    return gather_kernel


_gather = _make_gather()


def kernel(pos_encoding, t):
    idx = jnp.reshape(t.astype(jnp.int32), (_NW, _NCHUNK, _CHUNK))
    out = _gather(pos_encoding, idx)
    return jnp.reshape(out, (_B, _D))
